# baseline (speedup 1.0000x reference)
import jax
import jax.numpy as jnp
from jax import lax
from jax.experimental import pallas as pl
from jax.experimental.pallas import tpu as pltpu

N, D, H, C, K = 32768, 2048, 256, 16, 64
TOPK = 10
ROWS_PER_BLOCK = 1024
NUM_BLOCKS = N // ROWS_PER_BLOCK
SLOT = 128
NEG_INF = float("-inf")
BIG = 2**30


def _body(x_ref, w_ref, x_hbm, proto_ref, w2_ref, b2_ref, wr_ref, br_ref,
          wc_ref, bc_ref, bag_ref, prob_ref, yhat_ref, sim_ref,
          cand_s_v, cand_i_v, p_v, m_scratch, sem):
    i = pl.program_id(0)
    dn = (((1,), (1,)), ((), ()))

    @pl.when(i == 0)
    def _proto():
        p_v[...] = lax.dot_general(
            proto_ref[...], w2_ref[...], dn,
            preferred_element_type=jnp.float32) + b2_ref[...][None, :]

    @pl.when(i < NUM_BLOCKS)
    def _score():
        part = jnp.sum(x_ref[...] * w_ref[...][None, :], axis=1)
        sv = part.reshape(8, 128)
        lin = (lax.broadcasted_iota(jnp.int32, (8, 128), 0) * 128
               + lax.broadcasted_iota(jnp.int32, (8, 128), 1))
        slot_iota = lax.iota(jnp.int32, SLOT)
        sel_s = jnp.full((SLOT,), NEG_INF, jnp.float32)
        sel_i = jnp.full((SLOT,), BIG, jnp.int32)
        for t in range(TOPK):
            m = jnp.max(sv)
            idx = jnp.min(jnp.where(sv == m, lin, jnp.int32(ROWS_PER_BLOCK)))
            sel_s = jnp.where(slot_iota == t, m, sel_s)
            sel_i = jnp.where(slot_iota == t, i * ROWS_PER_BLOCK + idx, sel_i)
            sv = jnp.where(lin == idx, NEG_INF, sv)
        cand_s_v[i] = sel_s.reshape(1, SLOT)
        cand_i_v[i] = sel_i.reshape(1, SLOT)

    @pl.when(i == NUM_BLOCKS)
    def _tail():
        sv = cand_s_v[...].reshape(NUM_BLOCKS, SLOT)
        iv = cand_i_v[...].reshape(NUM_BLOCKS, SLOT)

        copies = []
        for t in range(TOPK):
            m = jnp.max(sv)
            g = jnp.min(jnp.where(sv == m, iv, BIG))
            cp = pltpu.make_async_copy(
                x_hbm.at[pl.ds(g, 1), :], m_scratch.at[pl.ds(t, 1), :], sem)
            cp.start()
            copies.append(cp)
            sv = jnp.where((sv == m) & (iv == g), NEG_INF, sv)
        for cp in copies:
            cp.wait()

        mrows = m_scratch[...]
        f = lax.dot_general(mrows, w2_ref[...], dn,
                            preferred_element_type=jnp.float32) + b2_ref[...][None, :]
        p = p_v[...]

        diff = f[:, None, :] - p[None, :, :] + 1e-6
        sim = jnp.sqrt(jnp.sum(diff * diff, axis=2))
        cmax = jnp.max(sim, axis=1, keepdims=True)
        sim = sim / cmax
        sim_coding = jnp.mean(sim, axis=0, keepdims=True)

        h = lax.dot_general(sim_coding, wr_ref[...], dn,
                            preferred_element_type=jnp.float32) + br_ref[...][None, :]
        h = jnp.maximum(h, 0.0)
        bag = lax.dot_general(h, wc_ref[...], dn,
                              preferred_element_type=jnp.float32) + bc_ref[...][None, :]
        prob = jax.nn.softmax(bag, axis=1)

        bag_ref[...] = bag
        prob_ref[...] = prob
        yhat_ref[...] = jnp.where(prob[:, 1:2] > prob[:, 0:1], 1, 0).astype(jnp.int32)
        sim_ref[...] = sim_coding


def kernel(x_path, prototype, W3, b3, W2, b2, Wr, br, Wc, bc):
    w = W3[1] - W3[0]
    out_shapes = (
        jax.ShapeDtypeStruct((1, 2), jnp.float32),
        jax.ShapeDtypeStruct((1, 2), jnp.float32),
        jax.ShapeDtypeStruct((1, 1), jnp.int32),
        jax.ShapeDtypeStruct((1, K), jnp.float32),
    )
    vmem = lambda: pl.BlockSpec(memory_space=pltpu.MemorySpace.VMEM)
    last = NUM_BLOCKS - 1
    bag, prob, yhat, sim_coding = pl.pallas_call(
        _body,
        grid=(NUM_BLOCKS + 1,),
        in_specs=[
            pl.BlockSpec((ROWS_PER_BLOCK, D),
                         lambda i: (jnp.minimum(i, last), 0)),
            pl.BlockSpec((D,), lambda i: (0,)),
            pl.BlockSpec(memory_space=pltpu.MemorySpace.HBM),
            vmem(), vmem(), vmem(), vmem(), vmem(), vmem(), vmem(),
        ],
        out_specs=(
            pl.BlockSpec((1, 2), lambda i: (0, 0)),
            pl.BlockSpec((1, 2), lambda i: (0, 0)),
            pl.BlockSpec((1, 1), lambda i: (0, 0)),
            pl.BlockSpec((1, K), lambda i: (0, 0)),
        ),
        out_shape=out_shapes,
        scratch_shapes=[
            pltpu.VMEM((NUM_BLOCKS, 1, SLOT), jnp.float32),
            pltpu.VMEM((NUM_BLOCKS, 1, SLOT), jnp.int32),
            pltpu.VMEM((K, H), jnp.float32),
            pltpu.VMEM((TOPK, D), jnp.float32),
            pltpu.SemaphoreType.DMA,
        ],
    )(x_path, w, x_path, prototype, W2, b2, Wr, br, Wc, bc)
    return (bag, prob, yhat.reshape(1), sim_coding)


# speedup vs baseline: 1.4260x; 1.4260x over previous
import jax
import jax.numpy as jnp
from jax import lax
from jax.experimental import pallas as pl
from jax.experimental.pallas import tpu as pltpu

N, D, H, C, K = 32768, 2048, 256, 16, 64
TOPK = 10
ROWS_PER_BLOCK = 512
NUM_BLOCKS = N // ROWS_PER_BLOCK
SUB = ROWS_PER_BLOCK // 128
NEG_INF = float("-inf")
BIG = 2**30


def _body(x_ref, w_ref, x_hbm, proto_ref, w2_ref, b2_ref, wr_ref, br_ref,
          wc_ref, bc_ref, bag_ref, prob_ref, yhat_ref, sim_ref,
          scores_v, lmax_v, p_v, m_scratch, sem):
    i = pl.program_id(0)
    dn = (((1,), (1,)), ((), ()))

    @pl.when(i == 0)
    def _proto():
        p_v[...] = lax.dot_general(
            proto_ref[...], w2_ref[...], dn,
            preferred_element_type=jnp.float32) + b2_ref[...][None, :]

    @pl.when(i < NUM_BLOCKS)
    def _score():
        part = jnp.sum(x_ref[...] * w_ref[...][None, :], axis=1)
        sv = part.reshape(SUB, 128)
        scores_v[i] = sv
        lmax_v[i] = jnp.max(sv, axis=0, keepdims=True)

    @pl.when(i == NUM_BLOCKS)
    def _tail():
        table = lmax_v[...].reshape(NUM_BLOCKS, 128)
        trow = lax.broadcasted_iota(jnp.int32, (NUM_BLOCKS, 128), 0)
        tcol = lax.broadcasted_iota(jnp.int32, (NUM_BLOCKS, 128), 1)
        brow = lax.broadcasted_iota(jnp.int32, (SUB, 128), 0)
        bcol = lax.broadcasted_iota(jnp.int32, (SUB, 128), 1)

        copies = []
        for t in range(TOPK):
            m = jnp.max(table)
            bl = jnp.min(jnp.where(table == m, trow * 128 + tcol, BIG))
            b = bl // 128
            blk = scores_v[b]
            sv = blk.reshape(SUB, 128)
            hit = sv == m
            g = b * ROWS_PER_BLOCK + jnp.min(
                jnp.where(hit, brow * 128 + bcol, BIG))
            cp = pltpu.make_async_copy(
                x_hbm.at[pl.ds(g, 1), :], m_scratch.at[pl.ds(t, 1), :], sem)
            cp.start()
            copies.append(cp)
            lpos = g - b * ROWS_PER_BLOCK
            sv = jnp.where(brow * 128 + bcol == lpos, NEG_INF, sv)
            scores_v[b] = sv
            lrow = jnp.max(sv, axis=0, keepdims=True)
            lmax_v[b] = lrow
            table = jnp.where(trow == b, lrow, table)
        for cp in copies:
            cp.wait()

        mrows = m_scratch[...]
        f = lax.dot_general(mrows, w2_ref[...], dn,
                            preferred_element_type=jnp.float32) + b2_ref[...][None, :]
        p = p_v[...]

        diff = f[:, None, :] - p[None, :, :] + 1e-6
        sim = jnp.sqrt(jnp.sum(diff * diff, axis=2))
        cmax = jnp.max(sim, axis=1, keepdims=True)
        sim = sim / cmax
        sim_coding = jnp.mean(sim, axis=0, keepdims=True)

        h = lax.dot_general(sim_coding, wr_ref[...], dn,
                            preferred_element_type=jnp.float32) + br_ref[...][None, :]
        h = jnp.maximum(h, 0.0)
        bag = lax.dot_general(h, wc_ref[...], dn,
                              preferred_element_type=jnp.float32) + bc_ref[...][None, :]
        prob = jax.nn.softmax(bag, axis=1)

        bag_ref[...] = bag
        prob_ref[...] = prob
        yhat_ref[...] = jnp.where(prob[:, 1:2] > prob[:, 0:1], 1, 0).astype(jnp.int32)
        sim_ref[...] = sim_coding


def kernel(x_path, prototype, W3, b3, W2, b2, Wr, br, Wc, bc):
    w = W3[1] - W3[0]
    out_shapes = (
        jax.ShapeDtypeStruct((1, 2), jnp.float32),
        jax.ShapeDtypeStruct((1, 2), jnp.float32),
        jax.ShapeDtypeStruct((1, 1), jnp.int32),
        jax.ShapeDtypeStruct((1, K), jnp.float32),
    )
    vmem = lambda: pl.BlockSpec(memory_space=pltpu.MemorySpace.VMEM)
    last = NUM_BLOCKS - 1
    bag, prob, yhat, sim_coding = pl.pallas_call(
        _body,
        grid=(NUM_BLOCKS + 1,),
        in_specs=[
            pl.BlockSpec((ROWS_PER_BLOCK, D),
                         lambda i: (jnp.minimum(i, last), 0)),
            pl.BlockSpec((D,), lambda i: (0,)),
            pl.BlockSpec(memory_space=pltpu.MemorySpace.HBM),
            vmem(), vmem(), vmem(), vmem(), vmem(), vmem(), vmem(),
        ],
        out_specs=(
            pl.BlockSpec((1, 2), lambda i: (0, 0)),
            pl.BlockSpec((1, 2), lambda i: (0, 0)),
            pl.BlockSpec((1, 1), lambda i: (0, 0)),
            pl.BlockSpec((1, K), lambda i: (0, 0)),
        ),
        out_shape=out_shapes,
        scratch_shapes=[
            pltpu.VMEM((NUM_BLOCKS, SUB, 128), jnp.float32),
            pltpu.VMEM((NUM_BLOCKS, 1, 128), jnp.float32),
            pltpu.VMEM((K, H), jnp.float32),
            pltpu.VMEM((TOPK, D), jnp.float32),
            pltpu.SemaphoreType.DMA,
        ],
    )(x_path, w, x_path, prototype, W2, b2, Wr, br, Wc, bc)
    return (bag, prob, yhat.reshape(1), sim_coding)


# speedup vs baseline: 1.5797x; 1.1078x over previous
import jax
import jax.numpy as jnp
from jax import lax
from jax.experimental import pallas as pl
from jax.experimental.pallas import tpu as pltpu

N, D, H, C, K = 32768, 2048, 256, 16, 64
TOPK = 10
ROWS_PER_BLOCK = 1024
NUM_BLOCKS = N // ROWS_PER_BLOCK
SUB = ROWS_PER_BLOCK // 128
NEG_INF = float("-inf")
BIG = 2**30


def _body(x_ref, w_ref, x_hbm, proto_ref, w2_ref, b2_ref, wr_ref, br_ref,
          wc_ref, bc_ref, bag_ref, prob_ref, yhat_ref, sim_ref,
          scores_v, lmax_v, p_v, m_scratch, sem):
    i = pl.program_id(0)
    dn = (((1,), (1,)), ((), ()))

    @pl.when(i == 0)
    def _proto():
        p_v[...] = lax.dot_general(
            proto_ref[...], w2_ref[...], dn,
            preferred_element_type=jnp.float32) + b2_ref[...][None, :]

    part = jnp.sum(x_ref[...] * w_ref[...][None, :], axis=1)
    sv0 = part.reshape(SUB, 128)
    scores_v[i] = sv0
    lmax_v[i] = jnp.max(sv0, axis=0, keepdims=True)

    @pl.when(i == NUM_BLOCKS - 1)
    def _tail():
        table = lmax_v[...].reshape(NUM_BLOCKS, 128)
        trow = lax.broadcasted_iota(jnp.int32, (NUM_BLOCKS, 128), 0)
        tcol = lax.broadcasted_iota(jnp.int32, (NUM_BLOCKS, 128), 1)
        brow = lax.broadcasted_iota(jnp.int32, (SUB, 128), 0)
        bcol = lax.broadcasted_iota(jnp.int32, (SUB, 128), 1)

        copies = []
        for t in range(TOPK):
            m = jnp.max(table)
            bl = jnp.min(jnp.where(table == m, trow * 128 + tcol, BIG))
            b = bl // 128
            blk = scores_v[b]
            sv = blk.reshape(SUB, 128)
            hit = sv == m
            g = b * ROWS_PER_BLOCK + jnp.min(
                jnp.where(hit, brow * 128 + bcol, BIG))
            cp = pltpu.make_async_copy(
                x_hbm.at[pl.ds(g, 1), :], m_scratch.at[pl.ds(t, 1), :], sem)
            cp.start()
            copies.append(cp)
            lpos = g - b * ROWS_PER_BLOCK
            sv = jnp.where(brow * 128 + bcol == lpos, NEG_INF, sv)
            scores_v[b] = sv
            lrow = jnp.max(sv, axis=0, keepdims=True)
            lmax_v[b] = lrow
            table = jnp.where(trow == b, lrow, table)
        for cp in copies:
            cp.wait()

        mrows = m_scratch[...]
        f = lax.dot_general(mrows, w2_ref[...], dn,
                            preferred_element_type=jnp.float32) + b2_ref[...][None, :]
        p = p_v[...]

        diff = f[:, None, :] - p[None, :, :] + 1e-6
        sim = jnp.sqrt(jnp.sum(diff * diff, axis=2))
        cmax = jnp.max(sim, axis=1, keepdims=True)
        sim = sim / cmax
        sim_coding = jnp.mean(sim, axis=0, keepdims=True)

        h = lax.dot_general(sim_coding, wr_ref[...], dn,
                            preferred_element_type=jnp.float32) + br_ref[...][None, :]
        h = jnp.maximum(h, 0.0)
        bag = lax.dot_general(h, wc_ref[...], dn,
                              preferred_element_type=jnp.float32) + bc_ref[...][None, :]
        prob = jax.nn.softmax(bag, axis=1)

        bag_ref[...] = bag
        prob_ref[...] = prob
        yhat_ref[...] = jnp.where(prob[:, 1:2] > prob[:, 0:1], 1, 0).astype(jnp.int32)
        sim_ref[...] = sim_coding


def kernel(x_path, prototype, W3, b3, W2, b2, Wr, br, Wc, bc):
    w = W3[1] - W3[0]
    out_shapes = (
        jax.ShapeDtypeStruct((1, 2), jnp.float32),
        jax.ShapeDtypeStruct((1, 2), jnp.float32),
        jax.ShapeDtypeStruct((1, 1), jnp.int32),
        jax.ShapeDtypeStruct((1, K), jnp.float32),
    )
    vmem = lambda: pl.BlockSpec(memory_space=pltpu.MemorySpace.VMEM)
    bag, prob, yhat, sim_coding = pl.pallas_call(
        _body,
        grid=(NUM_BLOCKS,),
        in_specs=[
            pl.BlockSpec((ROWS_PER_BLOCK, D), lambda i: (i, 0)),
            pl.BlockSpec((D,), lambda i: (0,)),
            pl.BlockSpec(memory_space=pltpu.MemorySpace.HBM),
            vmem(), vmem(), vmem(), vmem(), vmem(), vmem(), vmem(),
        ],
        out_specs=(
            pl.BlockSpec((1, 2), lambda i: (0, 0)),
            pl.BlockSpec((1, 2), lambda i: (0, 0)),
            pl.BlockSpec((1, 1), lambda i: (0, 0)),
            pl.BlockSpec((1, K), lambda i: (0, 0)),
        ),
        out_shape=out_shapes,
        scratch_shapes=[
            pltpu.VMEM((NUM_BLOCKS, SUB, 128), jnp.float32),
            pltpu.VMEM((NUM_BLOCKS, 1, 128), jnp.float32),
            pltpu.VMEM((K, H), jnp.float32),
            pltpu.VMEM((TOPK, D), jnp.float32),
            pltpu.SemaphoreType.DMA,
        ],
    )(x_path, w, x_path, prototype, W2, b2, Wr, br, Wc, bc)
    return (bag, prob, yhat.reshape(1), sim_coding)


# speedup vs baseline: 1.5834x; 1.0024x over previous
import jax
import jax.numpy as jnp
from jax import lax
from jax.experimental import pallas as pl
from jax.experimental.pallas import tpu as pltpu

N, D, H, C, K = 32768, 2048, 256, 16, 64
TOPK = 10
ROWS_PER_BLOCK = 1024
NUM_BLOCKS = N // ROWS_PER_BLOCK
SUB = ROWS_PER_BLOCK // 128
NEG_INF = float("-inf")
BIG = 2**30


def _body(x_ref, w_ref, x_hbm, proto_ref, w2_ref, b2_ref, wr_ref, br_ref,
          wc_ref, bc_ref, bag_ref, prob_ref, yhat_ref, sim_ref,
          scores_v, lmax_v, p_v, m_scratch, sem):
    i = pl.program_id(0)
    dn = (((1,), (1,)), ((), ()))

    @pl.when(i == 0)
    def _proto():
        p_v[...] = lax.dot_general(
            proto_ref[...], w2_ref[...], dn,
            preferred_element_type=jnp.float32) + b2_ref[...][None, :]

    @pl.when(i < NUM_BLOCKS)
    def _score():
        part = jnp.sum(x_ref[...] * w_ref[...][None, :], axis=1)
        sv = part.reshape(SUB, 128)
        scores_v[i] = sv
        lmax_v[i] = jnp.max(sv, axis=0, keepdims=True)

    @pl.when(i == NUM_BLOCKS)
    def _tail():
        table = lmax_v[...].reshape(NUM_BLOCKS, 128)
        trow = lax.broadcasted_iota(jnp.int32, (NUM_BLOCKS, 128), 0)
        tcol = lax.broadcasted_iota(jnp.int32, (NUM_BLOCKS, 128), 1)
        brow = lax.broadcasted_iota(jnp.int32, (SUB, 128), 0)
        bcol = lax.broadcasted_iota(jnp.int32, (SUB, 128), 1)

        copies = []
        for t in range(TOPK):
            m = jnp.max(table)
            bl = jnp.min(jnp.where(table == m, trow * 128 + tcol, BIG))
            b = bl // 128
            blk = scores_v[b]
            sv = blk.reshape(SUB, 128)
            hit = sv == m
            g = b * ROWS_PER_BLOCK + jnp.min(
                jnp.where(hit, brow * 128 + bcol, BIG))
            cp = pltpu.make_async_copy(
                x_hbm.at[pl.ds(g, 1), :], m_scratch.at[pl.ds(t, 1), :], sem)
            cp.start()
            copies.append(cp)
            lpos = g - b * ROWS_PER_BLOCK
            sv = jnp.where(brow * 128 + bcol == lpos, NEG_INF, sv)
            scores_v[b] = sv
            lrow = jnp.max(sv, axis=0, keepdims=True)
            lmax_v[b] = lrow
            table = jnp.where(trow == b, lrow, table)
        for cp in copies:
            cp.wait()

        mrows = m_scratch[...]
        f = lax.dot_general(mrows, w2_ref[...], dn,
                            preferred_element_type=jnp.float32) + b2_ref[...][None, :]
        p = p_v[...]

        diff = f[:, None, :] - p[None, :, :] + 1e-6
        sim = jnp.sqrt(jnp.sum(diff * diff, axis=2))
        cmax = jnp.max(sim, axis=1, keepdims=True)
        sim = sim / cmax
        sim_coding = jnp.mean(sim, axis=0, keepdims=True)

        h = lax.dot_general(sim_coding, wr_ref[...], dn,
                            preferred_element_type=jnp.float32) + br_ref[...][None, :]
        h = jnp.maximum(h, 0.0)
        bag = lax.dot_general(h, wc_ref[...], dn,
                              preferred_element_type=jnp.float32) + bc_ref[...][None, :]
        prob = jax.nn.softmax(bag, axis=1)

        bag_ref[...] = bag
        prob_ref[...] = prob
        yhat_ref[...] = jnp.where(prob[:, 1:2] > prob[:, 0:1], 1, 0).astype(jnp.int32)
        sim_ref[...] = sim_coding


def kernel(x_path, prototype, W3, b3, W2, b2, Wr, br, Wc, bc):
    w = W3[1] - W3[0]
    out_shapes = (
        jax.ShapeDtypeStruct((1, 2), jnp.float32),
        jax.ShapeDtypeStruct((1, 2), jnp.float32),
        jax.ShapeDtypeStruct((1, 1), jnp.int32),
        jax.ShapeDtypeStruct((1, K), jnp.float32),
    )
    vmem = lambda: pl.BlockSpec(memory_space=pltpu.MemorySpace.VMEM)
    bag, prob, yhat, sim_coding = pl.pallas_call(
        _body,
        grid=(NUM_BLOCKS + 1,),
        in_specs=[
            pl.BlockSpec((ROWS_PER_BLOCK, D),
                         lambda i: (jnp.minimum(i, NUM_BLOCKS - 1), 0)),
            pl.BlockSpec((D,), lambda i: (0,)),
            pl.BlockSpec(memory_space=pltpu.MemorySpace.HBM),
            vmem(), vmem(), vmem(), vmem(), vmem(), vmem(), vmem(),
        ],
        out_specs=(
            pl.BlockSpec((1, 2), lambda i: (0, 0)),
            pl.BlockSpec((1, 2), lambda i: (0, 0)),
            pl.BlockSpec((1, 1), lambda i: (0, 0)),
            pl.BlockSpec((1, K), lambda i: (0, 0)),
        ),
        out_shape=out_shapes,
        scratch_shapes=[
            pltpu.VMEM((NUM_BLOCKS, SUB, 128), jnp.float32),
            pltpu.VMEM((NUM_BLOCKS, 1, 128), jnp.float32),
            pltpu.VMEM((K, H), jnp.float32),
            pltpu.VMEM((TOPK, D), jnp.float32),
            pltpu.SemaphoreType.DMA,
        ],
    )(x_path, w, x_path, prototype, W2, b2, Wr, br, Wc, bc)
    return (bag, prob, yhat.reshape(1), sim_coding)
